# Initial kernel scaffold; baseline (speedup 1.0000x reference)
import jax
import jax.numpy as jnp
from jax.experimental import pallas as pl


def kernel(x, pos_num, table):
    raise NotImplementedError("write your pallas kernel here")


# speedup vs baseline: 3.3817x; 3.3817x over previous
import functools
import math

import jax
import jax.numpy as jnp
import numpy as np
from jax import lax
from jax.experimental import pallas as pl
from jax.experimental.pallas import tpu as pltpu
from jax.experimental.pallas import tpu_sc as plsc

_HIDDEN = 128
_MAX_LEN = 512


def _pe_table(d_model, max_len):
    pe = np.zeros((max_len, d_model), dtype=np.float32)
    position = np.arange(0, max_len, dtype=np.float32)[:, None]
    div_term = np.exp(
        np.arange(0, d_model, 2, dtype=np.float32) * -(math.log(10000.0) / d_model)
    )
    pe[:, 0::2] = np.sin(position * div_term)
    pe[:, 1::2] = np.cos(position * div_term)
    return pe


_PE = _pe_table(_HIDDEN, _MAX_LEN)

_NUM_CORES = 2
_NUM_SUBCORES = 16
_NW = _NUM_CORES * _NUM_SUBCORES
_LANES = 16


@functools.lru_cache(maxsize=None)
def _build(B, L, V, D):
    seqs_per_w = B // _NW
    c1 = min(L, 128)
    c2 = L - c1
    mesh = plsc.VectorSubcoreMesh(core_axis_name="c", subcore_axis_name="s")

    @functools.partial(
        pl.kernel,
        out_type=jax.ShapeDtypeStruct((B * L, D), jnp.float32),
        mesh=mesh,
        scratch_types=[
            pltpu.VMEM((L,), jnp.int32),
            pltpu.VMEM((L, D), jnp.float32),
            pltpu.VMEM((L, D), jnp.float32),
            pltpu.SemaphoreType.DMA,
        ],
    )
    def emb_kernel(x_hbm, pe_hbm, t_hbm, out_hbm, idx_v, rows_v, pe_v, sem):
        wid = lax.axis_index("s") * _NUM_CORES + lax.axis_index("c")
        pltpu.sync_copy(pe_hbm, pe_v)

        def seq_body(s, carry):
            base = (wid * seqs_per_w + s) * L
            pltpu.sync_copy(x_hbm.at[pl.ds(base, L)], idx_v)
            pltpu.async_copy(
                t_hbm.at[idx_v.at[pl.ds(0, c1)]], rows_v.at[pl.ds(0, c1)], sem
            ).wait()
            if c2:
                pltpu.async_copy(
                    t_hbm.at[idx_v.at[pl.ds(c1, c2)]], rows_v.at[pl.ds(c1, c2)], sem
                ).wait()

            def row_body(r, rc):
                for cc in range(D // _LANES):
                    sl = pl.ds(cc * _LANES, _LANES)
                    rows_v[r, sl] = rows_v[r, sl] + pe_v[r, sl]
                return rc

            lax.fori_loop(0, L, row_body, 0)
            pltpu.sync_copy(rows_v, out_hbm.at[pl.ds(base, L)])
            return carry

        lax.fori_loop(0, seqs_per_w, seq_body, 0)

    return emb_kernel


def kernel(x, pos_num, table):
    B, L = x.shape
    V, D = table.shape
    t = table.at[0].set(0.0)
    pe = jnp.asarray(_PE[:L])
    xf = x.reshape(B * L).astype(jnp.int32)
    out = _build(B, L, V, D)(xf, pe, t)
    return out.reshape(B, L, D)


# speedup vs baseline: 4.8047x; 1.4208x over previous
import functools
import math

import jax
import jax.numpy as jnp
import numpy as np
from jax import lax
from jax.experimental import pallas as pl
from jax.experimental.pallas import tpu as pltpu
from jax.experimental.pallas import tpu_sc as plsc

_HIDDEN = 128
_MAX_LEN = 512


def _pe_table(d_model, max_len):
    pe = np.zeros((max_len, d_model), dtype=np.float32)
    position = np.arange(0, max_len, dtype=np.float32)[:, None]
    div_term = np.exp(
        np.arange(0, d_model, 2, dtype=np.float32) * -(math.log(10000.0) / d_model)
    )
    pe[:, 0::2] = np.sin(position * div_term)
    pe[:, 1::2] = np.cos(position * div_term)
    return pe


_PE = _pe_table(_HIDDEN, _MAX_LEN)

_NUM_CORES = 2
_NUM_SUBCORES = 16
_NW = _NUM_CORES * _NUM_SUBCORES
_LANES = 16


@functools.lru_cache(maxsize=None)
def _build(B, L, V, D):
    nseq = B // _NW
    n_outer = nseq // 2
    c1 = min(L, 128)
    c2 = L - c1
    mesh = plsc.VectorSubcoreMesh(core_axis_name="c", subcore_axis_name="s")

    @functools.partial(
        pl.kernel,
        out_type=jax.ShapeDtypeStruct((B * L, D), jnp.float32),
        mesh=mesh,
        scratch_types=[
            pltpu.VMEM((L,), jnp.int32),
            pltpu.VMEM((L,), jnp.int32),
            pltpu.VMEM((L, D), jnp.float32),
            pltpu.VMEM((L, D), jnp.float32),
            pltpu.VMEM((L, D), jnp.float32),
            pltpu.SemaphoreType.DMA,
            pltpu.SemaphoreType.DMA,
            pltpu.SemaphoreType.DMA,
            pltpu.SemaphoreType.DMA,
        ],
    )
    def emb_kernel(
        x_hbm, pe_hbm, t_hbm, out_hbm,
        idx0, idx1, rows0, rows1, pe_v, gsem0, gsem1, ssem0, ssem1,
    ):
        wid = lax.axis_index("s") * _NUM_CORES + lax.axis_index("c")
        idxs = (idx0, idx1)
        rows = (rows0, rows1)
        gsems = (gsem0, gsem1)
        ssems = (ssem0, ssem1)
        base0 = wid * nseq * L

        def start_gather(buf, seq_base):
            pltpu.sync_copy(x_hbm.at[pl.ds(seq_base, L)], idxs[buf])
            pltpu.async_copy(
                t_hbm.at[idxs[buf].at[pl.ds(0, c1)]],
                rows[buf].at[pl.ds(0, c1)],
                gsems[buf],
            )
            pltpu.async_copy(
                t_hbm.at[idxs[buf].at[pl.ds(c1, c2)]],
                rows[buf].at[pl.ds(c1, c2)],
                gsems[buf],
            )

        def wait_gather(buf):
            pltpu.make_async_copy(
                t_hbm.at[idxs[buf].at[pl.ds(0, c1)]],
                rows[buf].at[pl.ds(0, c1)],
                gsems[buf],
            ).wait()
            pltpu.make_async_copy(
                t_hbm.at[idxs[buf].at[pl.ds(c1, c2)]],
                rows[buf].at[pl.ds(c1, c2)],
                gsems[buf],
            ).wait()

        def wait_store(buf, seq_base):
            pltpu.make_async_copy(
                rows[buf], out_hbm.at[pl.ds(seq_base, L)], ssems[buf]
            ).wait()

        pltpu.sync_copy(pe_hbm, pe_v)
        start_gather(0, base0)

        def body(i, carry):
            for b in range(2):
                s = 2 * i + b
                base = base0 + s * L
                nb = 1 - b
                if b == 0:

                    @pl.when(s > 0)
                    def _():
                        wait_store(nb, base - L)

                    start_gather(nb, base + L)
                else:

                    @pl.when(i < n_outer - 1)
                    def _():
                        wait_store(nb, base - L)
                        start_gather(nb, base + L)

                wait_gather(b)

                def row_body(r, rc):
                    for rr in range(2):
                        for cc in range(D // _LANES):
                            sl = pl.ds(cc * _LANES, _LANES)
                            plsc.addupdate(
                                rows[b].at[2 * r + rr, sl], pe_v[2 * r + rr, sl]
                            )
                    return rc

                lax.fori_loop(0, L // 2, row_body, 0)
                pltpu.async_copy(rows[b], out_hbm.at[pl.ds(base, L)], ssems[b])
            return carry

        lax.fori_loop(0, n_outer, body, 0)

        wait_store(0, base0 + (nseq - 2) * L)
        wait_store(1, base0 + (nseq - 1) * L)

    return emb_kernel


def kernel(x, pos_num, table):
    B, L = x.shape
    V, D = table.shape
    t = table.at[0].set(0.0)
    pe = jnp.asarray(_PE[:L])
    xf = x.reshape(B * L).astype(jnp.int32)
    out = _build(B, L, V, D)(xf, pe, t)
    return out.reshape(B, L, D)


# speedup vs baseline: 5.3462x; 1.1127x over previous
import functools
import math

import jax
import jax.numpy as jnp
import numpy as np
from jax import lax
from jax.experimental import pallas as pl
from jax.experimental.pallas import tpu as pltpu
from jax.experimental.pallas import tpu_sc as plsc

_HIDDEN = 128
_MAX_LEN = 512


def _pe_table(d_model, max_len):
    pe = np.zeros((max_len, d_model), dtype=np.float32)
    position = np.arange(0, max_len, dtype=np.float32)[:, None]
    div_term = np.exp(
        np.arange(0, d_model, 2, dtype=np.float32) * -(math.log(10000.0) / d_model)
    )
    pe[:, 0::2] = np.sin(position * div_term)
    pe[:, 1::2] = np.cos(position * div_term)
    return pe


_PE = _pe_table(_HIDDEN, _MAX_LEN)

_NUM_CORES = 2
_NUM_SUBCORES = 16
_NW = _NUM_CORES * _NUM_SUBCORES
_LANES = 16


@functools.lru_cache(maxsize=None)
def _build(B, L, V, D):
    nseq = B // _NW
    n_outer = nseq // 2
    c1 = min(L, 128)
    c2 = L - c1
    mesh = plsc.VectorSubcoreMesh(core_axis_name="c", subcore_axis_name="s")

    @functools.partial(
        pl.kernel,
        out_type=jax.ShapeDtypeStruct((B * L, D), jnp.float32),
        mesh=mesh,
        scratch_types=[
            pltpu.VMEM((L,), jnp.int32),
            pltpu.VMEM((L,), jnp.int32),
            pltpu.VMEM((L, D), jnp.float32),
            pltpu.VMEM((L, D), jnp.float32),
            pltpu.VMEM((L, D), jnp.float32),
            pltpu.VMEM_SHARED((V, D), jnp.float32),
            pltpu.SemaphoreType.DMA,
            pltpu.SemaphoreType.DMA,
            pltpu.SemaphoreType.DMA,
            pltpu.SemaphoreType.DMA,
        ],
    )
    def emb_kernel(
        x_hbm, pe_hbm, t_hbm, out_hbm,
        idx0, idx1, rows0, rows1, pe_v, t_sp, gsem0, gsem1, ssem0, ssem1,
    ):
        sid = lax.axis_index("s")
        wid = sid * _NUM_CORES + lax.axis_index("c")
        idxs = (idx0, idx1)
        rows = (rows0, rows1)
        gsems = (gsem0, gsem1)
        ssems = (ssem0, ssem1)
        base0 = wid * nseq * L

        def start_gather(buf, seq_base):
            pltpu.sync_copy(x_hbm.at[pl.ds(seq_base, L)], idxs[buf])
            pltpu.async_copy(
                t_sp.at[idxs[buf].at[pl.ds(0, c1)]],
                rows[buf].at[pl.ds(0, c1)],
                gsems[buf],
            )
            pltpu.async_copy(
                t_sp.at[idxs[buf].at[pl.ds(c1, c2)]],
                rows[buf].at[pl.ds(c1, c2)],
                gsems[buf],
            )

        def wait_gather(buf):
            pltpu.make_async_copy(
                t_sp.at[idxs[buf].at[pl.ds(0, c1)]],
                rows[buf].at[pl.ds(0, c1)],
                gsems[buf],
            ).wait()
            pltpu.make_async_copy(
                t_sp.at[idxs[buf].at[pl.ds(c1, c2)]],
                rows[buf].at[pl.ds(c1, c2)],
                gsems[buf],
            ).wait()

        def wait_store(buf, seq_base):
            pltpu.make_async_copy(
                rows[buf], out_hbm.at[pl.ds(seq_base, L)], ssems[buf]
            ).wait()

        n_stage = 5
        per_stage = V // n_stage

        @pl.when(sid < n_stage)
        def _():
            pltpu.sync_copy(
                t_hbm.at[pl.ds(sid * per_stage, per_stage)],
                t_sp.at[pl.ds(sid * per_stage, per_stage)],
            )

        pltpu.sync_copy(pe_hbm, pe_v)
        plsc.subcore_barrier()
        start_gather(0, base0)

        def body(i, carry):
            for b in range(2):
                s = 2 * i + b
                base = base0 + s * L
                nb = 1 - b
                if b == 0:

                    @pl.when(s > 0)
                    def _():
                        wait_store(nb, base - L)

                    start_gather(nb, base + L)
                else:

                    @pl.when(i < n_outer - 1)
                    def _():
                        wait_store(nb, base - L)
                        start_gather(nb, base + L)

                wait_gather(b)

                def row_body(r, rc):
                    for rr in range(2):
                        for cc in range(D // _LANES):
                            sl = pl.ds(cc * _LANES, _LANES)
                            plsc.addupdate(
                                rows[b].at[2 * r + rr, sl], pe_v[2 * r + rr, sl]
                            )
                    return rc

                lax.fori_loop(0, L // 2, row_body, 0)
                pltpu.async_copy(rows[b], out_hbm.at[pl.ds(base, L)], ssems[b])
            return carry

        lax.fori_loop(0, n_outer, body, 0)

        wait_store(0, base0 + (nseq - 2) * L)
        wait_store(1, base0 + (nseq - 1) * L)

    return emb_kernel


def kernel(x, pos_num, table):
    B, L = x.shape
    V, D = table.shape
    t = table.at[0].set(0.0)
    pe = jnp.asarray(_PE[:L])
    xf = x.reshape(B * L).astype(jnp.int32)
    out = _build(B, L, V, D)(xf, pe, t)
    return out.reshape(B, L, D)


# speedup vs baseline: 6.6135x; 1.2370x over previous
import functools
import math

import jax
import jax.numpy as jnp
import numpy as np
from jax import lax
from jax.experimental import pallas as pl
from jax.experimental.pallas import tpu as pltpu
from jax.experimental.pallas import tpu_sc as plsc

_HIDDEN = 128
_MAX_LEN = 512


def _pe_table(d_model, max_len):
    pe = np.zeros((max_len, d_model), dtype=np.float32)
    position = np.arange(0, max_len, dtype=np.float32)[:, None]
    div_term = np.exp(
        np.arange(0, d_model, 2, dtype=np.float32) * -(math.log(10000.0) / d_model)
    )
    pe[:, 0::2] = np.sin(position * div_term)
    pe[:, 1::2] = np.cos(position * div_term)
    return pe


_PE = _pe_table(_HIDDEN, _MAX_LEN)

_NUM_CORES = 2
_NUM_SUBCORES = 16
_NW = _NUM_CORES * _NUM_SUBCORES
_LANES = 16


@functools.lru_cache(maxsize=None)
def _build(B, L, V, D):
    nseq = B // _NW
    n_outer = nseq // 2
    c1 = min(L, 128)
    c2 = L - c1
    mesh = plsc.VectorSubcoreMesh(core_axis_name="c", subcore_axis_name="s")

    @functools.partial(
        pl.kernel,
        out_type=jax.ShapeDtypeStruct((B * L, D), jnp.float32),
        mesh=mesh,
        scratch_types=[
            pltpu.VMEM((L,), jnp.int32),
            pltpu.VMEM((L,), jnp.int32),
            pltpu.VMEM((L,), jnp.int32),
            pltpu.VMEM((L, D), jnp.float32),
            pltpu.VMEM((L, D), jnp.float32),
            pltpu.VMEM_SHARED((V, D), jnp.float32),
            pltpu.VMEM_SHARED((L, D), jnp.float32),
            pltpu.SemaphoreType.DMA,
            pltpu.SemaphoreType.DMA,
            pltpu.SemaphoreType.DMA,
            pltpu.SemaphoreType.DMA,
            pltpu.SemaphoreType.DMA,
            pltpu.SemaphoreType.DMA,
        ],
    )
    def emb_kernel(
        x_hbm, pe_hbm, t_hbm, iota_hbm, out_hbm,
        idx0, idx1, idx_pe, rows0, rows1, t_sp, pe_sp,
        gsem0, gsem1, asem0, asem1, ssem0, ssem1,
    ):
        sid = lax.axis_index("s")
        wid = sid * _NUM_CORES + lax.axis_index("c")
        idxs = (idx0, idx1)
        rows = (rows0, rows1)
        gsems = (gsem0, gsem1)
        asems = (asem0, asem1)
        ssems = (ssem0, ssem1)
        base0 = wid * nseq * L

        def chunks(buf, idx_ref, src, sem):
            return (
                pltpu.make_async_copy(
                    src.at[idx_ref.at[pl.ds(0, c1)]],
                    rows[buf].at[pl.ds(0, c1)],
                    sem,
                ),
                pltpu.make_async_copy(
                    src.at[idx_ref.at[pl.ds(c1, c2)]],
                    rows[buf].at[pl.ds(c1, c2)],
                    sem,
                ),
            )

        def start_gather(buf, seq_base):
            pltpu.sync_copy(x_hbm.at[pl.ds(seq_base, L)], idxs[buf])
            for c in chunks(buf, idxs[buf], t_sp, gsems[buf]):
                c.start()

        def wait_gather(buf):
            for c in chunks(buf, idxs[buf], t_sp, gsems[buf]):
                c.wait()

        def start_pe_add(buf):
            pltpu.async_copy(
                pe_sp.at[idx_pe.at[pl.ds(0, c1)]],
                rows[buf].at[pl.ds(0, c1)],
                asems[buf],
                add=True,
            )
            pltpu.async_copy(
                pe_sp.at[idx_pe.at[pl.ds(c1, c2)]],
                rows[buf].at[pl.ds(c1, c2)],
                asems[buf],
                add=True,
            )

        def wait_pe_add(buf):
            for c in chunks(buf, idx_pe, pe_sp, asems[buf]):
                c.wait()

        def wait_store(buf, seq_base):
            pltpu.make_async_copy(
                rows[buf], out_hbm.at[pl.ds(seq_base, L)], ssems[buf]
            ).wait()

        n_stage = 5
        per_stage = V // n_stage

        @pl.when(sid < n_stage)
        def _():
            pltpu.sync_copy(
                t_hbm.at[pl.ds(sid * per_stage, per_stage)],
                t_sp.at[pl.ds(sid * per_stage, per_stage)],
            )

        @pl.when(sid == n_stage)
        def _():
            pltpu.sync_copy(pe_hbm, pe_sp)

        pltpu.sync_copy(iota_hbm, idx_pe)
        plsc.subcore_barrier()
        start_gather(0, base0)

        def body(i, carry):
            for b in range(2):
                s = 2 * i + b
                base = base0 + s * L
                nb = 1 - b
                wait_gather(b)
                start_pe_add(b)
                if b == 0:

                    @pl.when(s > 0)
                    def _():
                        wait_store(nb, base - L)

                    start_gather(nb, base + L)
                else:

                    @pl.when(i < n_outer - 1)
                    def _():
                        wait_store(nb, base - L)
                        start_gather(nb, base + L)

                wait_pe_add(b)
                pltpu.async_copy(rows[b], out_hbm.at[pl.ds(base, L)], ssems[b])
            return carry

        lax.fori_loop(0, n_outer, body, 0)

        wait_store(0, base0 + (nseq - 2) * L)
        wait_store(1, base0 + (nseq - 1) * L)

    return emb_kernel


def kernel(x, pos_num, table):
    B, L = x.shape
    V, D = table.shape
    t = table.at[0].set(0.0)
    pe = jnp.asarray(_PE[:L])
    iota = jnp.arange(L, dtype=jnp.int32)
    xf = x.reshape(B * L).astype(jnp.int32)
    out = _build(B, L, V, D)(xf, pe, t, iota)
    return out.reshape(B, L, D)


# speedup vs baseline: 6.9286x; 1.0476x over previous
import functools
import math

import jax
import jax.numpy as jnp
import numpy as np
from jax import lax
from jax.experimental import pallas as pl
from jax.experimental.pallas import tpu as pltpu
from jax.experimental.pallas import tpu_sc as plsc

_HIDDEN = 128
_MAX_LEN = 512


def _pe_table(d_model, max_len):
    pe = np.zeros((max_len, d_model), dtype=np.float32)
    position = np.arange(0, max_len, dtype=np.float32)[:, None]
    div_term = np.exp(
        np.arange(0, d_model, 2, dtype=np.float32) * -(math.log(10000.0) / d_model)
    )
    pe[:, 0::2] = np.sin(position * div_term)
    pe[:, 1::2] = np.cos(position * div_term)
    return pe


_PE = _pe_table(_HIDDEN, _MAX_LEN)

_NUM_CORES = 2
_NUM_SUBCORES = 16
_NW = _NUM_CORES * _NUM_SUBCORES
_LANES = 16
_NBUF = 4


@functools.lru_cache(maxsize=None)
def _build(B, L, V, D):
    nseq = B // _NW
    n_outer = nseq // _NBUF
    c1 = min(L, 128)
    c2 = L - c1
    mesh = plsc.VectorSubcoreMesh(core_axis_name="c", subcore_axis_name="s")

    @functools.partial(
        pl.kernel,
        out_type=jax.ShapeDtypeStruct((B * L, D), jnp.float32),
        mesh=mesh,
        scratch_types=[
            [pltpu.VMEM((L,), jnp.int32)] * _NBUF,
            pltpu.VMEM((L,), jnp.int32),
            [pltpu.VMEM((L, D), jnp.float32)] * _NBUF,
            pltpu.VMEM_SHARED((V, D), jnp.float32),
            pltpu.VMEM_SHARED((L, D), jnp.float32),
            [pltpu.SemaphoreType.DMA] * _NBUF,
            [pltpu.SemaphoreType.DMA] * _NBUF,
            [pltpu.SemaphoreType.DMA] * _NBUF,
        ],
    )
    def emb_kernel(
        x_hbm, pe_hbm, t_hbm, iota_hbm, out_hbm,
        idxs, idx_pe, rows, t_sp, pe_sp, gsems, asems, ssems,
    ):
        sid = lax.axis_index("s")
        wid = sid * _NUM_CORES + lax.axis_index("c")
        base0 = wid * nseq * L

        def chunks(buf, idx_ref, src, sem):
            return (
                pltpu.make_async_copy(
                    src.at[idx_ref.at[pl.ds(0, c1)]],
                    rows[buf].at[pl.ds(0, c1)],
                    sem,
                ),
                pltpu.make_async_copy(
                    src.at[idx_ref.at[pl.ds(c1, c2)]],
                    rows[buf].at[pl.ds(c1, c2)],
                    sem,
                ),
            )

        def start_gather(buf, seq_base):
            pltpu.sync_copy(x_hbm.at[pl.ds(seq_base, L)], idxs[buf])
            for c in chunks(buf, idxs[buf], t_sp, gsems[buf]):
                c.start()

        def wait_gather(buf):
            for c in chunks(buf, idxs[buf], t_sp, gsems[buf]):
                c.wait()

        def start_pe_add(buf):
            pltpu.async_copy(
                pe_sp.at[idx_pe.at[pl.ds(0, c1)]],
                rows[buf].at[pl.ds(0, c1)],
                asems[buf],
                add=True,
            )
            pltpu.async_copy(
                pe_sp.at[idx_pe.at[pl.ds(c1, c2)]],
                rows[buf].at[pl.ds(c1, c2)],
                asems[buf],
                add=True,
            )

        def wait_pe_add(buf):
            for c in chunks(buf, idx_pe, pe_sp, asems[buf]):
                c.wait()

        def wait_store(buf, seq_base):
            pltpu.make_async_copy(
                rows[buf], out_hbm.at[pl.ds(seq_base, L)], ssems[buf]
            ).wait()

        n_stage = 5
        per_stage = V // n_stage

        @pl.when(sid < n_stage)
        def _():
            pltpu.sync_copy(
                t_hbm.at[pl.ds(sid * per_stage, per_stage)],
                t_sp.at[pl.ds(sid * per_stage, per_stage)],
            )

        @pl.when(sid == n_stage)
        def _():
            pltpu.sync_copy(pe_hbm, pe_sp)

        pltpu.sync_copy(iota_hbm, idx_pe)
        plsc.subcore_barrier()
        for b in range(_NBUF - 1):
            start_gather(b, base0 + b * L)

        def body(i, carry):
            for b in range(_NBUF):
                s = _NBUF * i + b
                base = base0 + s * L
                wait_gather(b)
                start_pe_add(b)
                pb = (b + _NBUF - 1) % _NBUF
                if b == 0:

                    @pl.when(s > 0)
                    def _():
                        wait_store(pb, base - L)

                    start_gather(pb, base + (_NBUF - 1) * L)
                else:

                    @pl.when(i < n_outer - 1)
                    def _():
                        wait_store(pb, base - L)
                        start_gather(pb, base + (_NBUF - 1) * L)

                wait_pe_add(b)
                pltpu.async_copy(rows[b], out_hbm.at[pl.ds(base, L)], ssems[b])
            return carry

        lax.fori_loop(0, n_outer, body, 0)

        for b in range(_NBUF):
            wait_store(b, base0 + (nseq - _NBUF + b) * L)

    return emb_kernel


def kernel(x, pos_num, table):
    B, L = x.shape
    V, D = table.shape
    t = table.at[0].set(0.0)
    pe = jnp.asarray(_PE[:L])
    iota = jnp.arange(L, dtype=jnp.int32)
    xf = x.reshape(B * L).astype(jnp.int32)
    out = _build(B, L, V, D)(xf, pe, t, iota)
    return out.reshape(B, L, D)
